# Initial kernel scaffold; baseline (speedup 1.0000x reference)
import jax
import jax.numpy as jnp
from jax.experimental import pallas as pl


def kernel(batch_ids, attention_mask, emb_weight):
    raise NotImplementedError("write your pallas kernel here")


# speedup vs baseline: 1.7799x; 1.7799x over previous
import functools

import jax
import jax.numpy as jnp
from jax import lax
from jax.experimental import pallas as pl
from jax.experimental.pallas import tpu as pltpu
from jax.experimental.pallas import tpu_sc as plsc

BATCH = 1024
SEQ = 200
EMB_DIM = 300

_B = BATCH * SEQ
_NW = 32
_B_PER_W = _B // _NW
_CHUNK = 128
_N_CHUNKS = _B_PER_W // _CHUNK
_BULK = 256
_TAIL = EMB_DIM - _BULK


def _sc_gather(emb_weight, idx_flat):
    mesh = plsc.VectorSubcoreMesh(core_axis_name="c", subcore_axis_name="s")

    @functools.partial(
        pl.kernel,
        out_type=jax.ShapeDtypeStruct((_B, EMB_DIM), jnp.float32),
        mesh=mesh,
        scratch_types=[
            pltpu.VMEM((_B_PER_W,), jnp.int32),
            pltpu.VMEM((_CHUNK, _BULK), jnp.float32),
            pltpu.VMEM((_CHUNK, _TAIL), jnp.float32),
            pltpu.SemaphoreType.DMA,
            pltpu.SemaphoreType.DMA,
            pltpu.SemaphoreType.DMA,
        ],
    )
    def k(table_hbm, idx_hbm, out_hbm, idx_v, bulk_v, tail_v, gsem, tsem, osem):
        nc = lax.axis_size("c")
        wid = lax.axis_index("s") * nc + lax.axis_index("c")
        base = wid * _B_PER_W
        pltpu.sync_copy(idx_hbm.at[pl.ds(base, _B_PER_W)], idx_v)

        def body(g, _):
            off = g * _CHUNK
            gather = pltpu.async_copy(
                table_hbm.at[idx_v.at[pl.ds(off, _CHUNK)], pl.ds(0, _BULK)],
                bulk_v,
                gsem,
            )
            tails = []
            for j16 in range(_CHUNK // 16):
                ids16 = idx_v[pl.ds(off + j16 * 16, 16)]
                for l in range(16):
                    j = j16 * 16 + l
                    tails.append(pltpu.async_copy(
                        table_hbm.at[ids16[l], pl.ds(_BULK, _TAIL)],
                        tail_v.at[j],
                        tsem,
                    ))
            gather.wait()
            for t in tails:
                t.wait()
            pltpu.async_copy(
                bulk_v,
                out_hbm.at[pl.ds(base + off, _CHUNK), pl.ds(0, _BULK)],
                osem,
            ).wait()
            pltpu.async_copy(
                tail_v,
                out_hbm.at[pl.ds(base + off, _CHUNK), pl.ds(_BULK, _TAIL)],
                osem,
            ).wait()
            return 0

        lax.fori_loop(0, _N_CHUNKS, body, 0)

    return k(emb_weight, idx_flat)


def kernel(batch_ids, attention_mask, emb_weight):
    idx_flat = batch_ids.reshape(_B)
    out = _sc_gather(emb_weight, idx_flat)
    return (out.reshape(BATCH, SEQ, EMB_DIM), attention_mask)


# speedup vs baseline: 1.9016x; 1.0683x over previous
import functools

import jax
import jax.numpy as jnp
from jax import lax
from jax.experimental import pallas as pl
from jax.experimental.pallas import tpu as pltpu
from jax.experimental.pallas import tpu_sc as plsc

BATCH = 1024
SEQ = 200
EMB_DIM = 300

_B = BATCH * SEQ
_NW = 32
_B_PER_W = _B // _NW
_CHUNK = 128
_N_CHUNKS = _B_PER_W // _CHUNK
_BULK = 256
_TAIL = EMB_DIM - _BULK


def _sc_gather(emb_weight, idx_flat):
    mesh = plsc.VectorSubcoreMesh(core_axis_name="c", subcore_axis_name="s")

    @functools.partial(
        pl.kernel,
        out_type=jax.ShapeDtypeStruct((_B, EMB_DIM), jnp.float32),
        mesh=mesh,
        scratch_types=[
            pltpu.VMEM((_B_PER_W,), jnp.int32),
            pltpu.VMEM((2, _CHUNK, _BULK), jnp.float32),
            pltpu.VMEM((2, _CHUNK, _TAIL), jnp.float32),
            pltpu.SemaphoreType.DMA,
            pltpu.SemaphoreType.DMA,
            pltpu.SemaphoreType.DMA,
            pltpu.SemaphoreType.DMA,
            pltpu.SemaphoreType.DMA,
            pltpu.SemaphoreType.DMA,
        ],
    )
    def k(table_hbm, idx_hbm, out_hbm, idx_v, bulk_v, tail_v,
          gsem0, gsem1, tsem0, tsem1, osem0, osem1):
        gsems = (gsem0, gsem1)
        tsems = (tsem0, tsem1)
        osems = (osem0, osem1)
        nc = lax.axis_size("c")
        wid = lax.axis_index("s") * nc + lax.axis_index("c")
        base = wid * _B_PER_W
        pltpu.sync_copy(idx_hbm.at[pl.ds(base, _B_PER_W)], idx_v)

        def fire_in(g, s):
            off = g * _CHUNK
            pltpu.async_copy(
                table_hbm.at[idx_v.at[pl.ds(off, _CHUNK)], pl.ds(0, _BULK)],
                bulk_v.at[s], gsems[s])
            for j16 in range(_CHUNK // 16):
                ids16 = idx_v[pl.ds(off + j16 * 16, 16)]
                for l in range(16):
                    pltpu.async_copy(
                        table_hbm.at[ids16[l], pl.ds(_BULK, _TAIL)],
                        tail_v.at[s].at[j16 * 16 + l], tsems[s])

        def wait_in(s):
            pltpu.make_async_copy(
                table_hbm.at[pl.ds(0, _CHUNK), pl.ds(0, _BULK)],
                bulk_v.at[s], gsems[s]).wait()
            pltpu.make_async_copy(
                table_hbm.at[pl.ds(0, _CHUNK), pl.ds(_BULK, _TAIL)],
                tail_v.at[s], tsems[s]).wait()

        def fire_out(g, s):
            off = g * _CHUNK
            pltpu.async_copy(
                bulk_v.at[s],
                out_hbm.at[pl.ds(base + off, _CHUNK), pl.ds(0, _BULK)],
                osems[s])
            pltpu.async_copy(
                tail_v.at[s],
                out_hbm.at[pl.ds(base + off, _CHUNK), pl.ds(_BULK, _TAIL)],
                osems[s])

        def wait_out(s):
            pltpu.make_async_copy(
                bulk_v.at[s],
                out_hbm.at[pl.ds(0, _CHUNK), pl.ds(0, _BULK)], osems[s]).wait()
            pltpu.make_async_copy(
                tail_v.at[s],
                out_hbm.at[pl.ds(0, _CHUNK), pl.ds(_BULK, _TAIL)],
                osems[s]).wait()

        fire_in(0, 0)

        def body(g2, _):
            ga = g2 * 2
            gb = ga + 1

            @pl.when(g2 > 0)
            def _():
                wait_out(1)

            fire_in(gb, 1)
            wait_in(0)
            fire_out(ga, 0)
            wait_out(0)

            @pl.when(g2 < _N_CHUNKS // 2 - 1)
            def _():
                fire_in(ga + 2, 0)

            wait_in(1)
            fire_out(gb, 1)
            return 0

        lax.fori_loop(0, _N_CHUNKS // 2, body, 0)
        wait_out(1)

    return k(emb_weight, idx_flat)


def kernel(batch_ids, attention_mask, emb_weight):
    idx_flat = batch_ids.reshape(_B)
    out = _sc_gather(emb_weight, idx_flat)
    return (out.reshape(BATCH, SEQ, EMB_DIM), attention_mask)
